# baseline (speedup 1.0000x reference)
import functools

import jax
import jax.numpy as jnp
from jax import lax
from jax.experimental import pallas as pl
from jax.experimental.pallas import tpu as pltpu
from jax.experimental.pallas import tpu_sc as plsc

_N = 100000
_D = 128
_CH = 160
_NCHUNKS = _N // _CH
_NW = 32
_L = 16
_KPW = -(-_NCHUNKS // _NW)


def _dl_body(x_hbm, m_hbm, w_hbm, o_hbm,
             xb0, xb1, mb0, mb1, ob0, ob1, mx0, mx1, wv,
             in_sem0, in_sem1, out_sem0, out_sem1):
    cid = lax.axis_index("c")
    sid = lax.axis_index("s")
    wid = sid * 2 + cid

    pltpu.sync_copy(w_hbm, wv)
    wregs = [wv[pl.ds(c * _L, _L)] for c in range(_D // _L)]

    xbs = (xb0, xb1)
    mxs = (mx0, mx1)
    mbs = (mb0, mb1)
    obs = (ob0, ob1)
    in_sems = (in_sem0, in_sem1)
    out_sems = (out_sem0, out_sem1)

    def x_in(k, slot):
        row0 = (wid + k * _NW) * _CH
        return pltpu.make_async_copy(
            x_hbm.at[pl.ds(row0, _CH), :], xbs[slot], in_sems[slot])

    def m_in(k, slot):
        e0 = (wid + k * _NW) * _CH
        return pltpu.make_async_copy(
            m_hbm.at[pl.ds(e0, _CH)], mbs[slot], in_sems[slot])

    def o_out(k, slot):
        row0 = (wid + k * _NW) * _CH
        return pltpu.make_async_copy(
            obs[slot], o_hbm.at[pl.ds(row0, _CH), :], out_sems[slot])

    def compute(slot):
        xs, ms, os_, mx = xbs[slot], mbs[slot], obs[slot], mxs[slot]

        @plsc.parallel_loop(0, _CH // _L, unroll=2)
        def expand_body(g):
            r0 = g * _L
            mvec = ms[pl.ds(r0, _L)]
            for lane in range(_L):
                mx[pl.ds((r0 + lane) * _L, _L)] = jnp.broadcast_to(
                    mvec[lane], (_L,))

        @plsc.parallel_loop(0, _CH, unroll=16)
        def row_body(r):
            sel = mx[pl.ds(r * _L, _L)] != 0.0
            for c in range(_D // _L):
                xv = xs[r, pl.ds(c * _L, _L)]
                os_[r, pl.ds(c * _L, _L)] = jnp.where(sel, xv * wregs[c], xv)

    x_in(0, 0).start()
    m_in(0, 0).start()
    x_in(1, 1).start()
    m_in(1, 1).start()
    x_in(0, 0).wait()
    m_in(0, 0).wait()
    compute(0)
    o_out(0, 0).start()

    def pair_body(i, _):
        ka = 1 + 2 * i
        kb = 2 + 2 * i

        x_in(ka + 1, 0).start()
        m_in(ka + 1, 0).start()
        x_in(ka, 1).wait()
        m_in(ka, 1).wait()

        @pl.when(ka >= 3)
        def _():
            o_out(ka - 2, 1).wait()

        compute(1)
        o_out(ka, 1).start()

        @pl.when(wid + (kb + 1) * _NW < _NCHUNKS)
        def _():
            x_in(kb + 1, 1).start()
            m_in(kb + 1, 1).start()

        x_in(kb, 0).wait()
        m_in(kb, 0).wait()
        o_out(kb - 2, 0).wait()
        compute(0)
        o_out(kb, 0).start()
        return 0

    lax.fori_loop(0, (_KPW - 1) // 2, pair_body, 0)

    last = _KPW - 1
    o_out(last - 2, 1).wait()
    o_out(last - 1, 0).wait()

    @pl.when(wid + last * _NW < _NCHUNKS)
    def _():
        x_in(last, 1).wait()
        m_in(last, 1).wait()
        compute(1)
        o_out(last, 1).start()
        o_out(last, 1).wait()


def kernel(x, node_mask, deletion_weight):
    n, d = x.shape
    mf = node_mask.astype(jnp.float32)
    mesh = plsc.VectorSubcoreMesh(core_axis_name="c", subcore_axis_name="s")
    f = functools.partial(
        pl.kernel,
        mesh=mesh,
        out_type=jax.ShapeDtypeStruct((n, d), x.dtype),
        scratch_types=[
            pltpu.VMEM((_CH, d), jnp.float32),
            pltpu.VMEM((_CH, d), jnp.float32),
            pltpu.VMEM((_CH,), jnp.float32),
            pltpu.VMEM((_CH,), jnp.float32),
            pltpu.VMEM((_CH, d), jnp.float32),
            pltpu.VMEM((_CH, d), jnp.float32),
            pltpu.VMEM((_CH * _L,), jnp.float32),
            pltpu.VMEM((_CH * _L,), jnp.float32),
            pltpu.VMEM((d,), jnp.float32),
            pltpu.SemaphoreType.DMA,
            pltpu.SemaphoreType.DMA,
            pltpu.SemaphoreType.DMA,
            pltpu.SemaphoreType.DMA,
        ],
    )(_dl_body)
    return f(x, mf, deletion_weight)


# speedup vs baseline: 1.0173x; 1.0173x over previous
import functools

import jax
import jax.numpy as jnp
from jax import lax
from jax.experimental import pallas as pl
from jax.experimental.pallas import tpu as pltpu
from jax.experimental.pallas import tpu_sc as plsc

_N = 100000
_D = 128
_CH = 160
_NCHUNKS = _N // _CH
_NW = 32
_L = 16
_KPW = -(-_NCHUNKS // _NW)


def _dl_body(x_hbm, m_hbm, w_hbm, o_hbm,
             xb0, xb1, mb0, mb1, ob0, ob1, mx0, mx1, wv,
             in_sem0, in_sem1, out_sem0, out_sem1):
    cid = lax.axis_index("c")
    sid = lax.axis_index("s")
    wid = sid * 2 + cid

    pltpu.sync_copy(w_hbm, wv)
    wregs = [wv[pl.ds(c * _L, _L)] for c in range(_D // _L)]

    xbs = (xb0, xb1)
    mxs = (mx0, mx1)
    mbs = (mb0, mb1)
    obs = (ob0, ob1)
    in_sems = (in_sem0, in_sem1)
    out_sems = (out_sem0, out_sem1)

    def x_in(k, slot):
        row0 = (wid + k * _NW) * _CH
        return pltpu.make_async_copy(
            x_hbm.at[pl.ds(row0, _CH), :], xbs[slot], in_sems[slot])

    def m_in(k, slot):
        e0 = (wid + k * _NW) * _CH
        return pltpu.make_async_copy(
            m_hbm.at[pl.ds(e0, _CH)], mbs[slot], in_sems[slot])

    def o_out(k, slot):
        row0 = (wid + k * _NW) * _CH
        return pltpu.make_async_copy(
            obs[slot], o_hbm.at[pl.ds(row0, _CH), :], out_sems[slot])

    def compute(slot):
        xs, ms, os_, mx = xbs[slot], mbs[slot], obs[slot], mxs[slot]

        @plsc.parallel_loop(0, _CH // _L, unroll=2)
        def expand_body(g):
            r0 = g * _L
            mvec = ms[pl.ds(r0, _L)]
            for lane in range(_L):
                mx[pl.ds((r0 + lane) * _L, _L)] = jnp.broadcast_to(
                    mvec[lane], (_L,))

        @plsc.parallel_loop(0, _CH, unroll=10)
        def row_body(r):
            sel = mx[pl.ds(r * _L, _L)] != 0.0
            for c in range(_D // _L):
                xv = xs[r, pl.ds(c * _L, _L)]
                os_[r, pl.ds(c * _L, _L)] = jnp.where(sel, xv * wregs[c], xv)

    x_in(0, 0).start()
    m_in(0, 0).start()
    x_in(1, 1).start()
    m_in(1, 1).start()
    x_in(0, 0).wait()
    m_in(0, 0).wait()
    compute(0)
    o_out(0, 0).start()

    def pair_body(i, _):
        ka = 1 + 2 * i
        kb = 2 + 2 * i

        x_in(ka + 1, 0).start()
        m_in(ka + 1, 0).start()
        x_in(ka, 1).wait()
        m_in(ka, 1).wait()

        @pl.when(ka >= 3)
        def _():
            o_out(ka - 2, 1).wait()

        compute(1)
        o_out(ka, 1).start()

        @pl.when(wid + (kb + 1) * _NW < _NCHUNKS)
        def _():
            x_in(kb + 1, 1).start()
            m_in(kb + 1, 1).start()

        x_in(kb, 0).wait()
        m_in(kb, 0).wait()
        o_out(kb - 2, 0).wait()
        compute(0)
        o_out(kb, 0).start()
        return 0

    lax.fori_loop(0, (_KPW - 1) // 2, pair_body, 0)

    last = _KPW - 1
    o_out(last - 2, 1).wait()
    o_out(last - 1, 0).wait()

    @pl.when(wid + last * _NW < _NCHUNKS)
    def _():
        x_in(last, 1).wait()
        m_in(last, 1).wait()
        compute(1)
        o_out(last, 1).start()
        o_out(last, 1).wait()


def kernel(x, node_mask, deletion_weight):
    n, d = x.shape
    mf = node_mask.astype(jnp.float32)
    mesh = plsc.VectorSubcoreMesh(core_axis_name="c", subcore_axis_name="s")
    f = functools.partial(
        pl.kernel,
        mesh=mesh,
        out_type=jax.ShapeDtypeStruct((n, d), x.dtype),
        scratch_types=[
            pltpu.VMEM((_CH, d), jnp.float32),
            pltpu.VMEM((_CH, d), jnp.float32),
            pltpu.VMEM((_CH,), jnp.float32),
            pltpu.VMEM((_CH,), jnp.float32),
            pltpu.VMEM((_CH, d), jnp.float32),
            pltpu.VMEM((_CH, d), jnp.float32),
            pltpu.VMEM((_CH * _L,), jnp.float32),
            pltpu.VMEM((_CH * _L,), jnp.float32),
            pltpu.VMEM((d,), jnp.float32),
            pltpu.SemaphoreType.DMA,
            pltpu.SemaphoreType.DMA,
            pltpu.SemaphoreType.DMA,
            pltpu.SemaphoreType.DMA,
        ],
    )(_dl_body)
    return f(x, mf, deletion_weight)


# speedup vs baseline: 1.0176x; 1.0003x over previous
import functools

import jax
import jax.numpy as jnp
from jax import lax
from jax.experimental import pallas as pl
from jax.experimental.pallas import tpu as pltpu
from jax.experimental.pallas import tpu_sc as plsc

_N = 100000
_D = 128
_CH = 200
_NCHUNKS = _N // _CH
_NW = 32
_L = 16
_KPW = -(-_NCHUNKS // _NW)


def _dl_body(x_hbm, m_hbm, w_hbm, o_hbm,
             xb0, xb1, mb0, mb1, ob0, ob1, mx0, mx1, wv,
             in_sem0, in_sem1, out_sem0, out_sem1):
    cid = lax.axis_index("c")
    sid = lax.axis_index("s")
    wid = sid * 2 + cid

    pltpu.sync_copy(w_hbm, wv)
    wregs = [wv[pl.ds(c * _L, _L)] for c in range(_D // _L)]

    xbs = (xb0, xb1)
    mxs = (mx0, mx1)
    mbs = (mb0, mb1)
    obs = (ob0, ob1)
    in_sems = (in_sem0, in_sem1)
    out_sems = (out_sem0, out_sem1)

    def x_in(k, slot):
        row0 = (wid + k * _NW) * _CH
        return pltpu.make_async_copy(
            x_hbm.at[pl.ds(row0, _CH), :], xbs[slot], in_sems[slot])

    def m_in(k, slot):
        e0 = (wid + k * _NW) * _CH
        return pltpu.make_async_copy(
            m_hbm.at[pl.ds(e0, _CH)], mbs[slot], in_sems[slot])

    def o_out(k, slot):
        row0 = (wid + k * _NW) * _CH
        return pltpu.make_async_copy(
            obs[slot], o_hbm.at[pl.ds(row0, _CH), :], out_sems[slot])

    def compute(slot):
        xs, ms, os_, mx = xbs[slot], mbs[slot], obs[slot], mxs[slot]

        @plsc.parallel_loop(0, _CH // _L, unroll=2)
        def expand_body(g):
            r0 = g * _L
            mvec = ms[pl.ds(r0, _L)]
            for lane in range(_L):
                mx[pl.ds((r0 + lane) * _L, _L)] = jnp.broadcast_to(
                    mvec[lane], (_L,))

        @plsc.parallel_loop(0, _CH, unroll=10)
        def row_body(r):
            sel = mx[pl.ds(r * _L, _L)] != 0.0
            for c in range(_D // _L):
                xv = xs[r, pl.ds(c * _L, _L)]
                os_[r, pl.ds(c * _L, _L)] = jnp.where(sel, xv * wregs[c], xv)

    x_in(0, 0).start()
    m_in(0, 0).start()
    x_in(1, 1).start()
    m_in(1, 1).start()
    x_in(0, 0).wait()
    m_in(0, 0).wait()
    compute(0)
    o_out(0, 0).start()

    def pair_body(i, _):
        ka = 1 + 2 * i
        kb = 2 + 2 * i

        x_in(ka + 1, 0).start()
        m_in(ka + 1, 0).start()
        x_in(ka, 1).wait()
        m_in(ka, 1).wait()

        @pl.when(ka >= 3)
        def _():
            o_out(ka - 2, 1).wait()

        compute(1)
        o_out(ka, 1).start()

        @pl.when(wid + (kb + 1) * _NW < _NCHUNKS)
        def _():
            x_in(kb + 1, 1).start()
            m_in(kb + 1, 1).start()

        x_in(kb, 0).wait()
        m_in(kb, 0).wait()
        o_out(kb - 2, 0).wait()
        compute(0)
        o_out(kb, 0).start()
        return 0

    lax.fori_loop(0, (_KPW - 1) // 2, pair_body, 0)

    last = _KPW - 1
    o_out(last - 2, 1).wait()
    o_out(last - 1, 0).wait()

    @pl.when(wid + last * _NW < _NCHUNKS)
    def _():
        x_in(last, 1).wait()
        m_in(last, 1).wait()
        compute(1)
        o_out(last, 1).start()
        o_out(last, 1).wait()


def kernel(x, node_mask, deletion_weight):
    n, d = x.shape
    mf = node_mask.astype(jnp.float32)
    mesh = plsc.VectorSubcoreMesh(core_axis_name="c", subcore_axis_name="s")
    f = functools.partial(
        pl.kernel,
        mesh=mesh,
        out_type=jax.ShapeDtypeStruct((n, d), x.dtype),
        scratch_types=[
            pltpu.VMEM((_CH, d), jnp.float32),
            pltpu.VMEM((_CH, d), jnp.float32),
            pltpu.VMEM((_CH,), jnp.float32),
            pltpu.VMEM((_CH,), jnp.float32),
            pltpu.VMEM((_CH, d), jnp.float32),
            pltpu.VMEM((_CH, d), jnp.float32),
            pltpu.VMEM((_CH * _L,), jnp.float32),
            pltpu.VMEM((_CH * _L,), jnp.float32),
            pltpu.VMEM((d,), jnp.float32),
            pltpu.SemaphoreType.DMA,
            pltpu.SemaphoreType.DMA,
            pltpu.SemaphoreType.DMA,
            pltpu.SemaphoreType.DMA,
        ],
    )(_dl_body)
    return f(x, mf, deletion_weight)


# speedup vs baseline: 1.0206x; 1.0029x over previous
import functools

import jax
import jax.numpy as jnp
from jax import lax
from jax.experimental import pallas as pl
from jax.experimental.pallas import tpu as pltpu
from jax.experimental.pallas import tpu_sc as plsc

_N = 100000
_D = 128
_CH = 160
_NCHUNKS = _N // _CH
_NW = 32
_L = 16
_KPW = -(-_NCHUNKS // _NW)


def _dl_body(x_hbm, m_hbm, w_hbm, o_hbm,
             xb0, xb1, mb0, mb1, ob0, ob1, mx0, mx1, wv,
             in_sem0, in_sem1, out_sem0, out_sem1):
    cid = lax.axis_index("c")
    sid = lax.axis_index("s")
    wid = sid * 2 + cid

    pltpu.sync_copy(w_hbm, wv)
    wregs = [wv[pl.ds(c * _L, _L)] for c in range(_D // _L)]

    xbs = (xb0, xb1)
    mxs = (mx0, mx1)
    mbs = (mb0, mb1)
    obs = (ob0, ob1)
    in_sems = (in_sem0, in_sem1)
    out_sems = (out_sem0, out_sem1)

    def x_in(k, slot):
        row0 = (wid + k * _NW) * _CH
        return pltpu.make_async_copy(
            x_hbm.at[pl.ds(row0, _CH), :], xbs[slot], in_sems[slot])

    def m_in(k, slot):
        e0 = (wid + k * _NW) * _CH
        return pltpu.make_async_copy(
            m_hbm.at[pl.ds(e0, _CH)], mbs[slot], in_sems[slot])

    def o_out(k, slot):
        row0 = (wid + k * _NW) * _CH
        return pltpu.make_async_copy(
            obs[slot], o_hbm.at[pl.ds(row0, _CH), :], out_sems[slot])

    def compute(slot):
        xs, ms, os_, mx = xbs[slot], mbs[slot], obs[slot], mxs[slot]

        @plsc.parallel_loop(0, _CH // _L, unroll=2)
        def expand_body(g):
            r0 = g * _L
            mvec = ms[pl.ds(r0, _L)]
            for lane in range(_L):
                mx[pl.ds((r0 + lane) * _L, _L)] = jnp.broadcast_to(
                    mvec[lane], (_L,))

        @plsc.parallel_loop(0, _CH, unroll=10)
        def row_body(r):
            sel = mx[pl.ds(r * _L, _L)] != 0.0
            for c in range(_D // _L):
                xv = xs[r, pl.ds(c * _L, _L)]
                os_[r, pl.ds(c * _L, _L)] = jnp.where(sel, xv * wregs[c], xv)

    x_in(0, 0).start()
    m_in(0, 0).start()
    x_in(1, 1).start()
    m_in(1, 1).start()
    x_in(0, 0).wait()
    m_in(0, 0).wait()
    compute(0)
    o_out(0, 0).start()

    def pair_body(i, _):
        ka = 1 + 2 * i
        kb = 2 + 2 * i

        x_in(ka + 1, 0).start()
        m_in(ka + 1, 0).start()
        x_in(ka, 1).wait()
        m_in(ka, 1).wait()

        @pl.when(ka >= 3)
        def _():
            o_out(ka - 2, 1).wait()

        compute(1)
        o_out(ka, 1).start()

        @pl.when(wid + (kb + 1) * _NW < _NCHUNKS)
        def _():
            x_in(kb + 1, 1).start()
            m_in(kb + 1, 1).start()

        x_in(kb, 0).wait()
        m_in(kb, 0).wait()
        o_out(kb - 2, 0).wait()
        compute(0)
        o_out(kb, 0).start()
        return 0

    lax.fori_loop(0, (_KPW - 1) // 2, pair_body, 0)

    last = _KPW - 1
    o_out(last - 2, 1).wait()
    o_out(last - 1, 0).wait()

    @pl.when(wid + last * _NW < _NCHUNKS)
    def _():
        x_in(last, 1).wait()
        m_in(last, 1).wait()
        compute(1)
        o_out(last, 1).start()
        o_out(last, 1).wait()


def kernel(x, node_mask, deletion_weight):
    n, d = x.shape
    mf = node_mask.astype(jnp.float32)
    mesh = plsc.VectorSubcoreMesh(core_axis_name="c", subcore_axis_name="s")
    f = functools.partial(
        pl.kernel,
        mesh=mesh,
        out_type=jax.ShapeDtypeStruct((n, d), x.dtype),
        scratch_types=[
            pltpu.VMEM((_CH, d), jnp.float32),
            pltpu.VMEM((_CH, d), jnp.float32),
            pltpu.VMEM((_CH,), jnp.float32),
            pltpu.VMEM((_CH,), jnp.float32),
            pltpu.VMEM((_CH, d), jnp.float32),
            pltpu.VMEM((_CH, d), jnp.float32),
            pltpu.VMEM((_CH * _L,), jnp.float32),
            pltpu.VMEM((_CH * _L,), jnp.float32),
            pltpu.VMEM((d,), jnp.float32),
            pltpu.SemaphoreType.DMA,
            pltpu.SemaphoreType.DMA,
            pltpu.SemaphoreType.DMA,
            pltpu.SemaphoreType.DMA,
        ],
    )(_dl_body)
    return f(x, mf, deletion_weight)
